# baseline (speedup 1.0000x reference)
import jax
import jax.numpy as jnp
from jax import lax
from jax.experimental import pallas as pl
from jax.experimental.pallas import tpu as pltpu
from jax.experimental.pallas import tpu_sc as plsc

N_IDENT = 10000
N_SYM = 10000
N_AST = 100000
N_OCC = 320000
D = 128

NC, NS = 2, 16
NW = NC * NS
CHUNK = 128
OCC_W = 10240
NCHUNK = OCC_W // CHUNK
OCC_RW = N_OCC // NW
SEG_SHIFT = 17
SYM_PAD = 12288
SYM_W = SYM_PAD // NW
SYM_CHUNKS = SYM_W // CHUNK
SEG_PAD = 10240
ROWS_T = SEG_PAD // NS
NCP = ROWS_T // CHUNK


def _sc_gather_segsum(ident_tab, sym_idx, ast_tab, packed_idx):
  mesh = plsc.VectorSubcoreMesh(
      core_axis_name="c", subcore_axis_name="s", num_cores=NC, num_subcores=NS)

  def body(ident_hbm, sym_hbm, ast_hbm, pidx_hbm, a_out, b_out,
           packed_v, symv, cidx, rows_a, rows_b, acc, sem_a, sem_b):
    c = lax.axis_index("c")
    s = lax.axis_index("s")
    wid = s * NC + c

    pltpu.sync_copy(pidx_hbm.at[wid], packed_v)
    pltpu.sync_copy(sym_hbm.at[wid], symv)

    bufs = ((rows_a, sem_a), (rows_b, sem_b))
    for k in range(SYM_CHUNKS):
      buf, sem = bufs[k % 2]
      if k < 2:
        pltpu.async_copy(ident_hbm.at[symv.at[k]], buf, sem)
      pltpu.make_async_copy(ident_hbm.at[symv.at[k]], buf, sem).wait()
      pltpu.sync_copy(buf, a_out.at[wid, pl.ds(k * CHUNK, CHUNK)])
      if k + 2 < SYM_CHUNKS:
        pltpu.async_copy(ident_hbm.at[symv.at[k + 2]], buf, sem)

    zero = jnp.zeros((16,), jnp.float32)

    @pl.loop(0, CHUNK)
    def _zero_rows(i):
      for j in range(D // 16):
        rows_a[i, pl.ds(j * 16, 16)] = zero

    for m in range(NCP):
      pltpu.sync_copy(rows_a, acc.at[pl.ds(s * ROWS_T + m * CHUNK, CHUNK)])

    def decode(j, t):
      for i in range(D // 16):
        v = packed_v[j, pl.ds(i * 16, 16)]
        cidx[t, 0, pl.ds(i * 16, 16)] = lax.bitwise_and(
            v, jnp.int32((1 << SEG_SHIFT) - 1))
        cidx[t, 1, pl.ds(i * 16, 16)] = lax.shift_right_logical(
            v, jnp.int32(SEG_SHIFT))

    def start_gather(t, buf, sem):
      pltpu.async_copy(ast_hbm.at[cidx.at[t, 0]], buf, sem)

    def wait_gather(t, buf, sem):
      pltpu.make_async_copy(ast_hbm.at[cidx.at[t, 0]], buf, sem).wait()

    decode(0, 0)
    decode(1, 1)
    start_gather(0, rows_a, sem_a)
    start_gather(1, rows_b, sem_b)
    plsc.subcore_barrier()

    @pl.loop(0, NCHUNK, step=2)
    def _chunk(j):
      wait_gather(0, rows_a, sem_a)
      pltpu.sync_copy(rows_a, acc.at[cidx.at[0, 1]], add=True)

      @pl.when(j + 2 < NCHUNK)
      def _():
        decode(j + 2, 0)
        start_gather(0, rows_a, sem_a)

      wait_gather(1, rows_b, sem_b)
      pltpu.sync_copy(rows_b, acc.at[cidx.at[1, 1]], add=True)

      @pl.when(j + 3 < NCHUNK)
      def _():
        decode(j + 3, 1)
        start_gather(1, rows_b, sem_b)

    plsc.subcore_barrier()

    for m in range(NCP):
      r0 = s * ROWS_T + m * CHUNK
      pltpu.sync_copy(acc.at[pl.ds(r0, CHUNK)], rows_a)
      pltpu.sync_copy(rows_a, b_out.at[c, pl.ds(r0, CHUNK)])

  f = pl.kernel(
      body,
      out_type=(
          jax.ShapeDtypeStruct((NW, SYM_W, D), jnp.float32),
          jax.ShapeDtypeStruct((NC, SEG_PAD, D), jnp.float32),
      ),
      mesh=mesh,
      scratch_types=(
          pltpu.VMEM((NCHUNK, CHUNK), jnp.int32),
          pltpu.VMEM((SYM_CHUNKS, CHUNK), jnp.int32),
          pltpu.VMEM((2, 2, CHUNK), jnp.int32),
          pltpu.VMEM((CHUNK, D), jnp.float32),
          pltpu.VMEM((CHUNK, D), jnp.float32),
          pltpu.VMEM_SHARED((SEG_PAD, D), jnp.float32),
          pltpu.SemaphoreType.DMA,
          pltpu.SemaphoreType.DMA,
      ),
  )
  return f(ident_tab, sym_idx, ast_tab, packed_idx)


BLK = 1000


def _tc_combine(a, b_partial, w1, w2):
  def body(a_ref, b_ref, w1_ref, w2_ref, o_ref):
    acc = jnp.dot(a_ref[...], w1_ref[...],
                  preferred_element_type=jnp.float32,
                  precision=lax.Precision.HIGHEST)
    acc = acc + jnp.dot(b_ref[0] + b_ref[1], w2_ref[...],
                        preferred_element_type=jnp.float32,
                        precision=lax.Precision.HIGHEST)
    o_ref[...] = jnp.maximum(acc, 0.0)

  return pl.pallas_call(
      body,
      grid=(N_SYM // BLK,),
      in_specs=[
          pl.BlockSpec((BLK, D), lambda i: (i, 0)),
          pl.BlockSpec((NC, BLK, D), lambda i: (0, i, 0)),
          pl.BlockSpec((D, D), lambda i: (0, 0)),
          pl.BlockSpec((D, D), lambda i: (0, 0)),
      ],
      out_specs=pl.BlockSpec((BLK, D), lambda i: (i, 0)),
      out_shape=jax.ShapeDtypeStruct((N_SYM, D), jnp.float32),
  )(a, b_partial, w1, w2)


def kernel(encoded_identifiers, symbols_identifier_indices, encoded_ast_nodes,
           ast_nodes_with_symbol_leaf_nodes_indices,
           ast_nodes_with_symbol_leaf_symbol_idx, W):
  sym_idx = symbols_identifier_indices.astype(jnp.int32)
  sym_idx = jnp.concatenate(
      [sym_idx, jnp.zeros((SYM_PAD - N_SYM,), jnp.int32)]
  ).reshape(NW, SYM_CHUNKS, CHUNK)

  node_idx = ast_nodes_with_symbol_leaf_nodes_indices.astype(jnp.int32)
  seg_idx = ast_nodes_with_symbol_leaf_symbol_idx.astype(jnp.int32)
  packed = jnp.bitwise_or(
      node_idx, jnp.left_shift(seg_idx, SEG_SHIFT)).reshape(NW, OCC_RW)
  pad = jnp.left_shift(N_SYM + jnp.arange(NW, dtype=jnp.int32),
                       SEG_SHIFT)[:, None]
  pad = jnp.broadcast_to(pad, (NW, OCC_W - OCC_RW))
  packed = jnp.concatenate([packed, pad], axis=1).reshape(NW, NCHUNK, CHUNK)

  a_gath, b_partial = _sc_gather_segsum(
      encoded_identifiers, sym_idx, encoded_ast_nodes, packed)
  a = a_gath.reshape(SYM_PAD, D)[:N_SYM]
  return _tc_combine(a, b_partial[:, :N_SYM], W[:D], W[D:])
